# baseline (speedup 1.0000x reference)
import functools

import jax
import jax.numpy as jnp
from jax import lax
from jax.experimental import pallas as pl
from jax.experimental.pallas import tpu as pltpu

B, A = 32, 64
N_SPECIES = 20
D = 64
K = 16
WIDTH = 0.5
RC = 5.0
UPDATE = 0.5
NCONV = 3
DECAY = 0.5
NS_PAD = 32


def _mol_kernel(spec_ref, pos_ref, post_ref, emb_ref, cw1_ref, cb1_ref,
                cw2_ref, cb2_ref, pw1_ref, pb1_ref, pw2_ref, pb2_ref,
                ew1_ref, eb1_ref, ew2_ref, eb2_ref,
                ciso_ref, cani_ref):
    f32 = jnp.float32

    srow = spec_ref[0]
    sb = jnp.broadcast_to(srow, (A, A))
    eye = (lax.broadcasted_iota(jnp.int32, (A, A), 0)
           == lax.broadcasted_iota(jnp.int32, (A, A), 1))
    scol = jnp.sum(jnp.where(eye, sb, 0), axis=1, keepdims=True)
    onehot = (lax.broadcasted_iota(jnp.int32, (A, NS_PAD), 1)
              == scol).astype(f32)
    f = jnp.dot(onehot, emb_ref[...], preferred_element_type=f32)

    P = pos_ref[0]
    PT = post_ref[0]
    r2 = jnp.zeros((A, A), f32)
    for c in range(3):
        d = P[:, c:c + 1] - PT[c:c + 1, :]
        r2 = r2 + d * d
    r = jnp.sqrt(r2 + 1e-9)

    rb = jnp.broadcast_to(r[None, :, :], (A, A, A))
    mT = (lax.broadcasted_iota(jnp.int32, (A, A, A), 0)
          == lax.broadcasted_iota(jnp.int32, (A, A, A), 2))
    R3 = jnp.sum(jnp.where(mT, rb, 0.0), axis=2, keepdims=True)

    i3 = lax.broadcasted_iota(jnp.int32, (A, A, 1), 0)
    j3 = lax.broadcasted_iota(jnp.int32, (A, A, 1), 1)
    ieqj = i3 == j3

    centers = (lax.broadcasted_iota(jnp.int32, (1, 1, K), 2).astype(f32)
               * (5.0 / (K - 1)))
    R16 = jnp.broadcast_to(R3, (A, A, K))
    dg = R16 - centers
    g = jnp.exp(dg * dg * (-1.0 / (2.0 * WIDTH * WIDTH)))
    g = jnp.where(ieqj, 0.0, g)
    g2 = g.reshape(A * A, K)
    h = jnp.tanh(jnp.dot(g2, cw1_ref[...], preferred_element_type=f32)
                 + cb1_ref[...])
    filt = (jnp.dot(h, cw2_ref[...], preferred_element_type=f32)
            + cb2_ref[...])
    filt3 = filt.reshape(A, A, D)

    for it in range(NCONV):
        fj = jnp.broadcast_to(f[None, :, :], (A, A, D))
        m = jnp.sum(filt3 * fj, axis=1)
        f = (1.0 - UPDATE) * f + (UPDATE * DECAY ** it) * jnp.tanh(m)

    ph = jnp.tanh(jnp.dot(f, pw1_ref[...], preferred_element_type=f32)
                  + pb1_ref[...])
    pa = jnp.sum(ph * pw2_ref[...], axis=1, keepdims=True) + pb2_ref[0, 0]
    ciso_ref[0] = jnp.sum(pa, axis=0, keepdims=True)

    fcut3 = (0.5 * (jnp.cos(R3 * (jnp.pi / RC)) + 1.0)
             * (R3 < RC).astype(f32))
    fcut3 = jnp.where(ieqj, 0.0, fcut3)
    fbb = jnp.broadcast_to(f[None, :, :], (A, A, D))
    fi3 = jnp.sum(jnp.where(ieqj, fbb, 0.0), axis=1,
                  keepdims=True)
    p3 = fi3 * fbb
    p2 = p3.reshape(A * A, D)
    eh = jnp.tanh(jnp.dot(p2, ew1_ref[...], preferred_element_type=f32)
                  + eb1_ref[...])
    e2 = jnp.sum(eh * ew2_ref[...], axis=1, keepdims=True) + eb2_ref[0, 0]
    e3 = e2.reshape(A, A, 1)
    cani_ref[0] = jnp.sum(e3 * fcut3, axis=1)


@jax.jit
def kernel(species, positions, emb, cw1, cb1, cw2, cb2, pw1, pb1, pw2, pb2,
           ew1, eb1, ew2, eb2):
    f32 = jnp.float32
    spec3 = species.astype(jnp.int32).reshape(B, 1, A)
    posp = jnp.pad(positions.astype(f32), ((0, 0), (0, 0), (0, 5)))
    post = jnp.transpose(posp, (0, 2, 1))
    embp = jnp.pad(emb, ((0, NS_PAD - N_SPECIES), (0, 0)))

    row = lambda v: v.reshape(1, -1)
    wvals = (embp, cw1, row(cb1), cw2, row(cb2), pw1, row(pb1),
             row(pw2), row(pb2), ew1, row(eb1), row(ew2), row(eb2))
    wspecs = [pl.BlockSpec(v.shape, lambda b: (0, 0)) for v in wvals]

    grid_spec = pl.GridSpec(
        grid=(B,),
        in_specs=[
            pl.BlockSpec((1, 1, A), lambda b: (b, 0, 0)),
            pl.BlockSpec((1, A, 8), lambda b: (b, 0, 0)),
            pl.BlockSpec((1, 8, A), lambda b: (b, 0, 0)),
        ] + wspecs,
        out_specs=[
            pl.BlockSpec((1, 1, 1), lambda b: (b, 0, 0)),
            pl.BlockSpec((1, A, 1), lambda b: (b, 0, 0)),
        ],
    )
    ciso3, cani = pl.pallas_call(
        _mol_kernel,
        grid_spec=grid_spec,
        out_shape=[
            jax.ShapeDtypeStruct((B, 1, 1), f32),
            jax.ShapeDtypeStruct((B, A, 1), f32),
        ],
        compiler_params=pltpu.CompilerParams(
            dimension_semantics=("parallel",)),
    )(spec3, posp, post, *wvals)
    return ciso3.reshape(B, 1), cani
